# baseline (speedup 1.0000x reference)
import functools

import jax
import jax.numpy as jnp
from jax import lax
from jax.experimental import pallas as pl
from jax.experimental.pallas import tpu as pltpu
from jax.experimental.pallas import tpu_sc as plsc

_N = 10000
_E = 160000
_NF = 16
_D = 32
_NG = 128
_NJ = 1024
_NS = 2048
_NOPS = 105

_NC = 2
_NSUB = 16
_NW = _NC * _NSUB
_CH = 128

_TE = 1000
_TN = 1000


def _lrelu(v):
    return jnp.where(v >= 0, v, 0.01 * v)


def _sc_gather(table, idx):
    n_rows = idx.shape[0]
    nch = n_rows // _CH
    per = (nch + _NW - 1) // _NW
    mesh = plsc.VectorSubcoreMesh(core_axis_name="c", subcore_axis_name="s")

    @functools.partial(
        pl.kernel,
        out_type=jax.ShapeDtypeStruct((n_rows, _D), jnp.float32),
        mesh=mesh,
        scratch_types=[
            pltpu.VMEM((_CH,), jnp.int32),
            pltpu.VMEM((_CH, _D), jnp.float32),
            pltpu.SemaphoreType.DMA,
        ],
        compiler_params=pltpu.CompilerParams(use_tc_tiling_on_sc=False),
    )
    def k(table_h, idx_h, out_h, idx_v, rows_v, sem):
        w = lax.axis_index("s") * _NC + lax.axis_index("c")

        def body(j, carry):
            ch = w + j * _NW

            @pl.when(ch < nch)
            def _():
                off = ch * _CH
                pltpu.sync_copy(idx_h.at[pl.ds(off, _CH)], idx_v)
                pltpu.async_copy(table_h.at[idx_v], rows_v, sem).wait()
                pltpu.sync_copy(rows_v, out_h.at[pl.ds(off, _CH)])

            return carry

        lax.fori_loop(0, per, body, 0)

    return k(table, idx)


def _sc_scatter(rows, idx, zeros_n):
    nch = rows.shape[0] // _CH
    hc = nch // _NC
    per = (hc + _NSUB - 1) // _NSUB
    zr = (_N // _NSUB) // 8 * 8
    rem = _N - zr * _NSUB
    mesh = plsc.VectorSubcoreMesh(core_axis_name="c", subcore_axis_name="s")

    @functools.partial(
        pl.kernel,
        out_type=jax.ShapeDtypeStruct((_NC, _N, _D), jnp.float32),
        mesh=mesh,
        scratch_types=[
            pltpu.VMEM((_CH,), jnp.int32),
            pltpu.VMEM((_CH, _D), jnp.float32),
            pltpu.VMEM_SHARED((_N, _D), jnp.float32),
        ],
        compiler_params=pltpu.CompilerParams(use_tc_tiling_on_sc=False),
    )
    def k(rows_h, idx_h, zeros_h, out_h, idx_v, rows_v, agg_sh):
        c = lax.axis_index("c")
        s = lax.axis_index("s")
        pltpu.sync_copy(zeros_h.at[pl.ds(s * zr, zr)], agg_sh.at[pl.ds(s * zr, zr)])

        @pl.when(s == _NSUB - 1)
        def _():
            pltpu.sync_copy(zeros_h.at[pl.ds(zr * _NSUB, rem)],
                            agg_sh.at[pl.ds(zr * _NSUB, rem)])

        plsc.subcore_barrier()

        def body(j, carry):
            local = s + j * _NSUB

            @pl.when(local < hc)
            def _():
                off = (c * hc + local) * _CH
                pltpu.sync_copy(idx_h.at[pl.ds(off, _CH)], idx_v)
                pltpu.sync_copy(rows_h.at[pl.ds(off, _CH)], rows_v)
                pltpu.sync_copy(rows_v, agg_sh.at[idx_v], add=True)

            return carry

        lax.fori_loop(0, per, body, 0)
        plsc.subcore_barrier()
        pltpu.sync_copy(agg_sh.at[pl.ds(s * zr, zr)], out_h.at[c].at[pl.ds(s * zr, zr)])

        @pl.when(s == _NSUB - 1)
        def _():
            pltpu.sync_copy(agg_sh.at[pl.ds(zr * _NSUB, rem)],
                            out_h.at[c].at[pl.ds(zr * _NSUB, rem)])

    return k(rows, idx, zeros_n)


def _tc_prologue(x, W0, b0, cntp):
    grid = _N // _TN

    def body(x_ref, w_ref, b_ref, cp_ref, out_ref, ci_ref):
        o = jax.lax.dot_general(
            x_ref[...], w_ref[...], (((1,), (0,)), ((), ())),
            preferred_element_type=jnp.float32) + b_ref[...]
        out_ref[...] = _lrelu(o)
        cnt = jnp.maximum(cp_ref[0] + cp_ref[1], 1.0)
        ci_ref[...] = 1.0 / cnt

    return pl.pallas_call(
        body,
        grid=(grid,),
        in_specs=[
            pl.BlockSpec((_TN, _NF), lambda i: (i, 0)),
            pl.BlockSpec((_NF, _D), lambda i: (0, 0)),
            pl.BlockSpec((1, _D), lambda i: (0, 0)),
            pl.BlockSpec((_NC, _TN, _D), lambda i: (0, i, 0)),
        ],
        out_specs=[
            pl.BlockSpec((_TN, _D), lambda i: (i, 0)),
            pl.BlockSpec((_TN, _D), lambda i: (i, 0)),
        ],
        out_shape=[
            jax.ShapeDtypeStruct((_N, _D), jnp.float32),
            jax.ShapeDtypeStruct((_N, _D), jnp.float32),
        ],
    )(x, W0, b0.reshape(1, _D), cntp)


def _tc_msg(edge_attr, s, A1, c1, A2f, c2r):
    grid = _E // _TE

    def body(ea_ref, s_ref, a1_ref, c1_ref, a2_ref, c2_ref, out_ref):
        sv = s_ref[...]
        he = jax.lax.dot_general(
            ea_ref[...], a1_ref[...], (((1,), (0,)), ((), ())),
            preferred_element_type=jnp.float32) + c1_ref[...]
        he = _lrelu(he)
        u = jnp.concatenate([he[:, k:k + 1] * sv for k in range(_D)], axis=1)
        acc = jax.lax.dot_general(
            u, a2_ref[...], (((1,), (0,)), ((), ())),
            preferred_element_type=jnp.float32)
        acc = acc + jax.lax.dot_general(
            sv, c2_ref[...], (((1,), (0,)), ((), ())),
            preferred_element_type=jnp.float32)
        out_ref[...] = acc

    return pl.pallas_call(
        body,
        grid=(grid,),
        in_specs=[
            pl.BlockSpec((_TE, 4), lambda i: (i, 0)),
            pl.BlockSpec((_TE, _D), lambda i: (i, 0)),
            pl.BlockSpec((4, _D), lambda i: (0, 0)),
            pl.BlockSpec((1, _D), lambda i: (0, 0)),
            pl.BlockSpec((_D * _D, _D), lambda i: (0, 0)),
            pl.BlockSpec((_D, _D), lambda i: (0, 0)),
        ],
        out_specs=pl.BlockSpec((_TE, _D), lambda i: (i, 0)),
        out_shape=jax.ShapeDtypeStruct((_E, _D), jnp.float32),
    )(edge_attr, s, A1, c1.reshape(1, _D), A2f, c2r)


def _tc_update(aggp, cnt_inv, state, root, cb, Wih, Whh, bih, bhh):
    grid = _N // _TN

    def body(ap_ref, ci_ref, st_ref, root_ref, cb_ref, wih_ref, whh_ref,
             bih_ref, bhh_ref, out_ref):
        st = st_ref[...]
        agg = (ap_ref[0] + ap_ref[1]) * ci_ref[...]
        m = agg + jax.lax.dot_general(
            st, root_ref[...], (((1,), (0,)), ((), ())),
            preferred_element_type=jnp.float32) + cb_ref[...]
        m = _lrelu(m)
        gi = jax.lax.dot_general(
            m, wih_ref[...], (((1,), (0,)), ((), ())),
            preferred_element_type=jnp.float32) + bih_ref[...]
        gh = jax.lax.dot_general(
            st, whh_ref[...], (((1,), (0,)), ((), ())),
            preferred_element_type=jnp.float32) + bhh_ref[...]
        r = jax.nn.sigmoid(gi[:, :_D] + gh[:, :_D])
        z = jax.nn.sigmoid(gi[:, _D:2 * _D] + gh[:, _D:2 * _D])
        n = jnp.tanh(gi[:, 2 * _D:] + r * gh[:, 2 * _D:])
        out_ref[...] = (1.0 - z) * n + z * st

    return pl.pallas_call(
        body,
        grid=(grid,),
        in_specs=[
            pl.BlockSpec((_NC, _TN, _D), lambda i: (0, i, 0)),
            pl.BlockSpec((_TN, _D), lambda i: (i, 0)),
            pl.BlockSpec((_TN, _D), lambda i: (i, 0)),
            pl.BlockSpec((_D, _D), lambda i: (0, 0)),
            pl.BlockSpec((1, _D), lambda i: (0, 0)),
            pl.BlockSpec((_D, 3 * _D), lambda i: (0, 0)),
            pl.BlockSpec((_D, 3 * _D), lambda i: (0, 0)),
            pl.BlockSpec((1, 3 * _D), lambda i: (0, 0)),
            pl.BlockSpec((1, 3 * _D), lambda i: (0, 0)),
        ],
        out_specs=pl.BlockSpec((_TN, _D), lambda i: (i, 0)),
        out_shape=jax.ShapeDtypeStruct((_N, _D), jnp.float32),
    )(aggp, cnt_inv, state, root, cb.reshape(1, _D), Wih, Whh,
      bih.reshape(1, 3 * _D), bhh.reshape(1, 3 * _D))


def _hl_from_biases(bi, bh):
    g = bi + bh
    i_g = g[:, :_D]
    g_g = g[:, 2 * _D:3 * _D]
    o_g = g[:, 3 * _D:]
    cl = jax.nn.sigmoid(i_g) * jnp.tanh(g_g)
    return jax.nn.sigmoid(o_g) * jnp.tanh(cl)


def _tc_emax(state, batch2, bih, bhh):
    grid = _N // _TN

    def body(st_ref, b_ref, bi_ref, bh_ref, out_ref):
        i = pl.program_id(0)
        hl = _hl_from_biases(bi_ref[...], bh_ref[...])
        e = jnp.sum(st_ref[...] * hl, axis=1, keepdims=True)
        m = b_ref[...] == lax.broadcasted_iota(jnp.int32, (_TN, _NG), 1)
        vals = jnp.max(jnp.where(m, e, -1e30), axis=0, keepdims=True)

        @pl.when(i == 0)
        def _():
            out_ref[...] = jnp.full((1, _NG), -1e30, jnp.float32)

        out_ref[...] = jnp.maximum(out_ref[...], vals)

    return pl.pallas_call(
        body,
        grid=(grid,),
        in_specs=[
            pl.BlockSpec((_TN, _D), lambda i: (i, 0)),
            pl.BlockSpec((_TN, 1), lambda i: (i, 0)),
            pl.BlockSpec((1, 4 * _D), lambda i: (0, 0)),
            pl.BlockSpec((1, 4 * _D), lambda i: (0, 0)),
        ],
        out_specs=pl.BlockSpec((1, _NG), lambda i: (0, 0)),
        out_shape=jax.ShapeDtypeStruct((1, _NG), jnp.float32),
    )(state, batch2, bih, bhh)


def _tc_denom(state, batch2, bih, bhh, emax):
    grid = _N // _TN

    def body(st_ref, b_ref, bi_ref, bh_ref, em_ref, out_ref):
        i = pl.program_id(0)
        hl = _hl_from_biases(bi_ref[...], bh_ref[...])
        e = jnp.sum(st_ref[...] * hl, axis=1, keepdims=True)
        m = b_ref[...] == lax.broadcasted_iota(jnp.int32, (_TN, _NG), 1)
        em_n = jnp.sum(jnp.where(m, em_ref[...], 0.0), axis=1, keepdims=True)
        ex = jnp.exp(e - em_n)
        vals = jnp.sum(jnp.where(m, ex, 0.0), axis=0, keepdims=True)

        @pl.when(i == 0)
        def _():
            out_ref[...] = jnp.zeros((1, _NG), jnp.float32)

        out_ref[...] = out_ref[...] + vals

    return pl.pallas_call(
        body,
        grid=(grid,),
        in_specs=[
            pl.BlockSpec((_TN, _D), lambda i: (i, 0)),
            pl.BlockSpec((_TN, 1), lambda i: (i, 0)),
            pl.BlockSpec((1, 4 * _D), lambda i: (0, 0)),
            pl.BlockSpec((1, 4 * _D), lambda i: (0, 0)),
            pl.BlockSpec((1, _NG), lambda i: (0, 0)),
        ],
        out_specs=pl.BlockSpec((1, _NG), lambda i: (0, 0)),
        out_shape=jax.ShapeDtypeStruct((1, _NG), jnp.float32),
    )(state, batch2, bih, bhh, emax)


def _tc_rvec(state, batch2, bih, bhh, emax, denom):
    grid = _N // _TN

    def body(st_ref, b_ref, bi_ref, bh_ref, em_ref, dn_ref, out_ref):
        i = pl.program_id(0)
        hl = _hl_from_biases(bi_ref[...], bh_ref[...])
        st = st_ref[...]
        e = jnp.sum(st * hl, axis=1, keepdims=True)
        m = b_ref[...] == lax.broadcasted_iota(jnp.int32, (_TN, _NG), 1)
        em_n = jnp.sum(jnp.where(m, em_ref[...], 0.0), axis=1, keepdims=True)
        dn_n = jnp.sum(jnp.where(m, dn_ref[...], 0.0), axis=1, keepdims=True)
        a = jnp.exp(e - em_n) / dn_n
        mf = m.astype(jnp.float32)
        vals = jax.lax.dot_general(
            mf, a * st, (((0,), (0,)), ((), ())),
            preferred_element_type=jnp.float32)

        @pl.when(i == 0)
        def _():
            out_ref[...] = jnp.zeros((_NG, _D), jnp.float32)

        out_ref[...] = out_ref[...] + vals

    return pl.pallas_call(
        body,
        grid=(grid,),
        in_specs=[
            pl.BlockSpec((_TN, _D), lambda i: (i, 0)),
            pl.BlockSpec((_TN, 1), lambda i: (i, 0)),
            pl.BlockSpec((1, 4 * _D), lambda i: (0, 0)),
            pl.BlockSpec((1, 4 * _D), lambda i: (0, 0)),
            pl.BlockSpec((1, _NG), lambda i: (0, 0)),
            pl.BlockSpec((1, _NG), lambda i: (0, 0)),
        ],
        out_specs=pl.BlockSpec((_NG, _D), lambda i: (0, 0)),
        out_shape=jax.ShapeDtypeStruct((_NG, _D), jnp.float32),
    )(state, batch2, bih, bhh, emax, denom)


def _tc_final(rvec, ghead, J1, jb1, J2, jb2, S1, sb1, S2, sb2,
              bih, bhh, Wout, bout):
    def body(rv_ref, gh_ref, j1_ref, jb1_ref, j2_ref, jb2_ref,
             s1_ref, sb1_ref, s2_ref, sb2_ref, bi_ref, bh_ref,
             wo_ref, bo_ref, fin_ref, jb_ref, st_ref):
        hl = _hl_from_biases(bi_ref[...], bh_ref[...])
        q = jnp.broadcast_to(hl, (_NG, _D))
        qstar = jnp.concatenate([q, rv_ref[...]], axis=1)
        fin_ref[...] = jax.lax.dot_general(
            qstar, wo_ref[...], (((1,), (0,)), ((), ())),
            preferred_element_type=jnp.float32) + bo_ref[...]
        gj = gh_ref[:2 * _NJ]
        t = _lrelu(jax.lax.dot_general(
            gj, j1_ref[...], (((1,), (0,)), ((), ())),
            preferred_element_type=jnp.float32) + jb1_ref[...])
        t = jax.lax.dot_general(
            t, j2_ref[...], (((1,), (0,)), ((), ())),
            preferred_element_type=jnp.float32) + jb2_ref[...]
        jb_ref[...] = 0.5 * (t[:_NJ] + t[_NJ:])
        gs = gh_ref[2 * _NJ:]
        u = _lrelu(jax.lax.dot_general(
            gs, s1_ref[...], (((1,), (0,)), ((), ())),
            preferred_element_type=jnp.float32) + sb1_ref[...])
        st_ref[...] = jax.lax.dot_general(
            u, s2_ref[...], (((1,), (0,)), ((), ())),
            preferred_element_type=jnp.float32) + sb2_ref[...]

    return pl.pallas_call(
        body,
        out_shape=[
            jax.ShapeDtypeStruct((_NG, 2), jnp.float32),
            jax.ShapeDtypeStruct((_NJ, 1), jnp.float32),
            jax.ShapeDtypeStruct((_NS, _NOPS), jnp.float32),
        ],
    )(rvec, ghead, J1, jb1.reshape(1, _D), J2, jb2.reshape(1, 1),
      S1, sb1.reshape(1, _D), S2, sb2.reshape(1, _NOPS),
      bih, bhh, Wout, bout.reshape(1, 2))


def kernel(x, edge_index, edge_attr, jbond_atmidx, stem_atmidx, batch,
           W0, b0, A1, c1, A2, c2, root, cb,
           gru_Wih, gru_Whh, gru_bih, gru_bhh,
           S1, sb1, S2, sb2, J1, jb1, J2, jb2,
           lstm_Wih, lstm_Whh, lstm_bih, lstm_bhh, Wout, bout):
    src = edge_index[0]
    dst = edge_index[1]
    A2f = A2.reshape(_D * _D, _D)
    c2r = c2.reshape(_D, _D)
    batch2 = batch.reshape(_N, 1)
    lbih = lstm_bih.reshape(1, 4 * _D)
    lbhh = lstm_bhh.reshape(1, 4 * _D)
    zeros_n = jnp.zeros((_N, _D), jnp.float32)
    ones_e = jnp.ones((_E, _D), jnp.float32)

    cntp = _sc_scatter(ones_e, dst, zeros_n)
    state, cnt_inv = _tc_prologue(x, W0, b0, cntp)

    for _ in range(6):
        s = _sc_gather(state, src)
        msg = _tc_msg(edge_attr, s, A1, c1, A2f, c2r)
        aggp = _sc_scatter(msg, dst, zeros_n)
        state = _tc_update(aggp, cnt_inv, state, root, cb,
                           gru_Wih, gru_Whh, gru_bih, gru_bhh)

    hidx = jnp.concatenate(
        [jbond_atmidx[:, 0], jbond_atmidx[:, 1], stem_atmidx])
    ghead = _sc_gather(state, hidx)

    emax = _tc_emax(state, batch2, lbih, lbhh)
    denom = _tc_denom(state, batch2, lbih, lbhh, emax)
    rvec = _tc_rvec(state, batch2, lbih, lbhh, emax, denom)
    final, jb, stem = _tc_final(rvec, ghead, J1, jb1, J2, jb2,
                                S1, sb1, S2, sb2, lbih, lbhh, Wout, bout)
    return final, jb.reshape(_NJ), stem


# speedup vs baseline: 2.0792x; 2.0792x over previous
import functools

import jax
import jax.numpy as jnp
from jax import lax
from jax.experimental import pallas as pl
from jax.experimental.pallas import tpu as pltpu
from jax.experimental.pallas import tpu_sc as plsc

_N = 10000
_E = 160000
_NF = 16
_D = 32
_NG = 128
_NJ = 1024
_NS = 2048
_NOPS = 105

_NC = 2
_NSUB = 16
_NW = _NC * _NSUB
_CH = 128

_TE = 1000
_TN = 1000


def _lrelu(v):
    return jnp.where(v >= 0, v, 0.01 * v)


def _sc_gather(table, idx):
    n_rows = idx.shape[0]
    nch = n_rows // _CH
    per = (nch + _NW - 1) // _NW
    mesh = plsc.VectorSubcoreMesh(core_axis_name="c", subcore_axis_name="s")

    @functools.partial(
        pl.kernel,
        out_type=jax.ShapeDtypeStruct((n_rows, _D), jnp.float32),
        mesh=mesh,
        scratch_types=[
            pltpu.VMEM((_CH,), jnp.int32),
            pltpu.VMEM((_CH, _D), jnp.float32),
            pltpu.SemaphoreType.DMA,
        ],
        compiler_params=pltpu.CompilerParams(use_tc_tiling_on_sc=False),
    )
    def k(table_h, idx_h, out_h, idx_v, rows_v, sem):
        w = lax.axis_index("s") * _NC + lax.axis_index("c")

        def body(j, carry):
            ch = w + j * _NW

            @pl.when(ch < nch)
            def _():
                off = ch * _CH
                pltpu.sync_copy(idx_h.at[pl.ds(off, _CH)], idx_v)
                pltpu.async_copy(table_h.at[idx_v], rows_v, sem).wait()
                pltpu.sync_copy(rows_v, out_h.at[pl.ds(off, _CH)])

            return carry

        lax.fori_loop(0, per, body, 0)

    return k(table, idx)


def _sc_scatter(rows, idx, zeros_n):
    nch = rows.shape[0] // _CH
    hc = nch // _NC
    per = (hc + _NSUB - 1) // _NSUB
    zr = (_N // _NSUB) // 8 * 8
    rem = _N - zr * _NSUB
    mesh = plsc.VectorSubcoreMesh(core_axis_name="c", subcore_axis_name="s")

    @functools.partial(
        pl.kernel,
        out_type=jax.ShapeDtypeStruct((_NC, _N, _D), jnp.float32),
        mesh=mesh,
        scratch_types=[
            pltpu.VMEM((_CH,), jnp.int32),
            pltpu.VMEM((_CH, _D), jnp.float32),
            pltpu.VMEM_SHARED((_N, _D), jnp.float32),
        ],
        compiler_params=pltpu.CompilerParams(use_tc_tiling_on_sc=False),
    )
    def k(rows_h, idx_h, zeros_h, out_h, idx_v, rows_v, agg_sh):
        c = lax.axis_index("c")
        s = lax.axis_index("s")
        pltpu.sync_copy(zeros_h.at[pl.ds(s * zr, zr)], agg_sh.at[pl.ds(s * zr, zr)])

        @pl.when(s == _NSUB - 1)
        def _():
            pltpu.sync_copy(zeros_h.at[pl.ds(zr * _NSUB, rem)],
                            agg_sh.at[pl.ds(zr * _NSUB, rem)])

        plsc.subcore_barrier()

        def body(j, carry):
            local = s + j * _NSUB

            @pl.when(local < hc)
            def _():
                off = (c * hc + local) * _CH
                pltpu.sync_copy(idx_h.at[pl.ds(off, _CH)], idx_v)
                pltpu.sync_copy(rows_h.at[pl.ds(off, _CH)], rows_v)
                pltpu.sync_copy(rows_v, agg_sh.at[idx_v], add=True)

            return carry

        lax.fori_loop(0, per, body, 0)
        plsc.subcore_barrier()
        pltpu.sync_copy(agg_sh.at[pl.ds(s * zr, zr)], out_h.at[c].at[pl.ds(s * zr, zr)])

        @pl.when(s == _NSUB - 1)
        def _():
            pltpu.sync_copy(agg_sh.at[pl.ds(zr * _NSUB, rem)],
                            out_h.at[c].at[pl.ds(zr * _NSUB, rem)])

    return k(rows, idx, zeros_n)


def _tc_prologue(x, W0, b0, cntp):
    grid = _N // _TN

    def body(x_ref, w_ref, b_ref, cp_ref, out_ref, ci_ref):
        o = jax.lax.dot_general(
            x_ref[...], w_ref[...], (((1,), (0,)), ((), ())),
            preferred_element_type=jnp.float32) + b_ref[...]
        out_ref[...] = _lrelu(o)
        cnt = jnp.maximum(cp_ref[0] + cp_ref[1], 1.0)
        ci_ref[...] = 1.0 / cnt

    return pl.pallas_call(
        body,
        grid=(grid,),
        in_specs=[
            pl.BlockSpec((_TN, _NF), lambda i: (i, 0)),
            pl.BlockSpec((_NF, _D), lambda i: (0, 0)),
            pl.BlockSpec((1, _D), lambda i: (0, 0)),
            pl.BlockSpec((_NC, _TN, _D), lambda i: (0, i, 0)),
        ],
        out_specs=[
            pl.BlockSpec((_TN, _D), lambda i: (i, 0)),
            pl.BlockSpec((_TN, _D), lambda i: (i, 0)),
        ],
        out_shape=[
            jax.ShapeDtypeStruct((_N, _D), jnp.float32),
            jax.ShapeDtypeStruct((_N, _D), jnp.float32),
        ],
    )(x, W0, b0.reshape(1, _D), cntp)


def _tc_msg(edge_attr, s, A1, c1, A2f, c2r, rep_he, rep_s):
    grid = _E // _TE

    def body(ea_ref, s_ref, a1_ref, c1_ref, a2_ref, c2_ref, rh_ref, rs_ref,
             out_ref):
        sv = s_ref[...]
        he = jax.lax.dot_general(
            ea_ref[...], a1_ref[...], (((1,), (0,)), ((), ())),
            preferred_element_type=jnp.float32) + c1_ref[...]
        he = _lrelu(he).astype(jnp.bfloat16)
        sv16 = sv.astype(jnp.bfloat16)
        he_rep = jax.lax.dot_general(
            he, rh_ref[...], (((1,), (0,)), ((), ())),
            preferred_element_type=jnp.float32)
        s_rep = jax.lax.dot_general(
            sv16, rs_ref[...], (((1,), (0,)), ((), ())),
            preferred_element_type=jnp.float32)
        u = (he_rep * s_rep).astype(jnp.bfloat16)
        acc = jax.lax.dot_general(
            u, a2_ref[...], (((1,), (0,)), ((), ())),
            preferred_element_type=jnp.float32)
        acc = acc + jax.lax.dot_general(
            sv16, c2_ref[...], (((1,), (0,)), ((), ())),
            preferred_element_type=jnp.float32)
        out_ref[...] = acc

    return pl.pallas_call(
        body,
        grid=(grid,),
        in_specs=[
            pl.BlockSpec((_TE, 4), lambda i: (i, 0)),
            pl.BlockSpec((_TE, _D), lambda i: (i, 0)),
            pl.BlockSpec((4, _D), lambda i: (0, 0)),
            pl.BlockSpec((1, _D), lambda i: (0, 0)),
            pl.BlockSpec((_D * _D, _D), lambda i: (0, 0)),
            pl.BlockSpec((_D, _D), lambda i: (0, 0)),
            pl.BlockSpec((_D, _D * _D), lambda i: (0, 0)),
            pl.BlockSpec((_D, _D * _D), lambda i: (0, 0)),
        ],
        out_specs=pl.BlockSpec((_TE, _D), lambda i: (i, 0)),
        out_shape=jax.ShapeDtypeStruct((_E, _D), jnp.float32),
    )(edge_attr, s, A1, c1.reshape(1, _D), A2f, c2r, rep_he, rep_s)


def _tc_update(aggp, cnt_inv, state, root, cb, Wih, Whh, bih, bhh):
    grid = _N // _TN

    def body(ap_ref, ci_ref, st_ref, root_ref, cb_ref, wih_ref, whh_ref,
             bih_ref, bhh_ref, out_ref):
        st = st_ref[...]
        agg = (ap_ref[0] + ap_ref[1]) * ci_ref[...]
        m = agg + jax.lax.dot_general(
            st, root_ref[...], (((1,), (0,)), ((), ())),
            preferred_element_type=jnp.float32) + cb_ref[...]
        m = _lrelu(m)
        gi = jax.lax.dot_general(
            m, wih_ref[...], (((1,), (0,)), ((), ())),
            preferred_element_type=jnp.float32) + bih_ref[...]
        gh = jax.lax.dot_general(
            st, whh_ref[...], (((1,), (0,)), ((), ())),
            preferred_element_type=jnp.float32) + bhh_ref[...]
        r = jax.nn.sigmoid(gi[:, :_D] + gh[:, :_D])
        z = jax.nn.sigmoid(gi[:, _D:2 * _D] + gh[:, _D:2 * _D])
        n = jnp.tanh(gi[:, 2 * _D:] + r * gh[:, 2 * _D:])
        out_ref[...] = (1.0 - z) * n + z * st

    return pl.pallas_call(
        body,
        grid=(grid,),
        in_specs=[
            pl.BlockSpec((_NC, _TN, _D), lambda i: (0, i, 0)),
            pl.BlockSpec((_TN, _D), lambda i: (i, 0)),
            pl.BlockSpec((_TN, _D), lambda i: (i, 0)),
            pl.BlockSpec((_D, _D), lambda i: (0, 0)),
            pl.BlockSpec((1, _D), lambda i: (0, 0)),
            pl.BlockSpec((_D, 3 * _D), lambda i: (0, 0)),
            pl.BlockSpec((_D, 3 * _D), lambda i: (0, 0)),
            pl.BlockSpec((1, 3 * _D), lambda i: (0, 0)),
            pl.BlockSpec((1, 3 * _D), lambda i: (0, 0)),
        ],
        out_specs=pl.BlockSpec((_TN, _D), lambda i: (i, 0)),
        out_shape=jax.ShapeDtypeStruct((_N, _D), jnp.float32),
    )(aggp, cnt_inv, state, root, cb.reshape(1, _D), Wih, Whh,
      bih.reshape(1, 3 * _D), bhh.reshape(1, 3 * _D))


def _hl_from_biases(bi, bh):
    g = bi + bh
    i_g = g[:, :_D]
    g_g = g[:, 2 * _D:3 * _D]
    o_g = g[:, 3 * _D:]
    cl = jax.nn.sigmoid(i_g) * jnp.tanh(g_g)
    return jax.nn.sigmoid(o_g) * jnp.tanh(cl)


def _tc_emax(state, batch2, bih, bhh):
    grid = _N // _TN

    def body(st_ref, b_ref, bi_ref, bh_ref, out_ref):
        i = pl.program_id(0)
        hl = _hl_from_biases(bi_ref[...], bh_ref[...])
        e = jnp.sum(st_ref[...] * hl, axis=1, keepdims=True)
        m = b_ref[...] == lax.broadcasted_iota(jnp.int32, (_TN, _NG), 1)
        vals = jnp.max(jnp.where(m, e, -1e30), axis=0, keepdims=True)

        @pl.when(i == 0)
        def _():
            out_ref[...] = jnp.full((1, _NG), -1e30, jnp.float32)

        out_ref[...] = jnp.maximum(out_ref[...], vals)

    return pl.pallas_call(
        body,
        grid=(grid,),
        in_specs=[
            pl.BlockSpec((_TN, _D), lambda i: (i, 0)),
            pl.BlockSpec((_TN, 1), lambda i: (i, 0)),
            pl.BlockSpec((1, 4 * _D), lambda i: (0, 0)),
            pl.BlockSpec((1, 4 * _D), lambda i: (0, 0)),
        ],
        out_specs=pl.BlockSpec((1, _NG), lambda i: (0, 0)),
        out_shape=jax.ShapeDtypeStruct((1, _NG), jnp.float32),
    )(state, batch2, bih, bhh)


def _tc_denom(state, batch2, bih, bhh, emax):
    grid = _N // _TN

    def body(st_ref, b_ref, bi_ref, bh_ref, em_ref, out_ref):
        i = pl.program_id(0)
        hl = _hl_from_biases(bi_ref[...], bh_ref[...])
        e = jnp.sum(st_ref[...] * hl, axis=1, keepdims=True)
        m = b_ref[...] == lax.broadcasted_iota(jnp.int32, (_TN, _NG), 1)
        em_n = jnp.sum(jnp.where(m, em_ref[...], 0.0), axis=1, keepdims=True)
        ex = jnp.exp(e - em_n)
        vals = jnp.sum(jnp.where(m, ex, 0.0), axis=0, keepdims=True)

        @pl.when(i == 0)
        def _():
            out_ref[...] = jnp.zeros((1, _NG), jnp.float32)

        out_ref[...] = out_ref[...] + vals

    return pl.pallas_call(
        body,
        grid=(grid,),
        in_specs=[
            pl.BlockSpec((_TN, _D), lambda i: (i, 0)),
            pl.BlockSpec((_TN, 1), lambda i: (i, 0)),
            pl.BlockSpec((1, 4 * _D), lambda i: (0, 0)),
            pl.BlockSpec((1, 4 * _D), lambda i: (0, 0)),
            pl.BlockSpec((1, _NG), lambda i: (0, 0)),
        ],
        out_specs=pl.BlockSpec((1, _NG), lambda i: (0, 0)),
        out_shape=jax.ShapeDtypeStruct((1, _NG), jnp.float32),
    )(state, batch2, bih, bhh, emax)


def _tc_rvec(state, batch2, bih, bhh, emax, denom):
    grid = _N // _TN

    def body(st_ref, b_ref, bi_ref, bh_ref, em_ref, dn_ref, out_ref):
        i = pl.program_id(0)
        hl = _hl_from_biases(bi_ref[...], bh_ref[...])
        st = st_ref[...]
        e = jnp.sum(st * hl, axis=1, keepdims=True)
        m = b_ref[...] == lax.broadcasted_iota(jnp.int32, (_TN, _NG), 1)
        em_n = jnp.sum(jnp.where(m, em_ref[...], 0.0), axis=1, keepdims=True)
        dn_n = jnp.sum(jnp.where(m, dn_ref[...], 0.0), axis=1, keepdims=True)
        a = jnp.exp(e - em_n) / dn_n
        mf = m.astype(jnp.float32)
        vals = jax.lax.dot_general(
            mf, a * st, (((0,), (0,)), ((), ())),
            preferred_element_type=jnp.float32)

        @pl.when(i == 0)
        def _():
            out_ref[...] = jnp.zeros((_NG, _D), jnp.float32)

        out_ref[...] = out_ref[...] + vals

    return pl.pallas_call(
        body,
        grid=(grid,),
        in_specs=[
            pl.BlockSpec((_TN, _D), lambda i: (i, 0)),
            pl.BlockSpec((_TN, 1), lambda i: (i, 0)),
            pl.BlockSpec((1, 4 * _D), lambda i: (0, 0)),
            pl.BlockSpec((1, 4 * _D), lambda i: (0, 0)),
            pl.BlockSpec((1, _NG), lambda i: (0, 0)),
            pl.BlockSpec((1, _NG), lambda i: (0, 0)),
        ],
        out_specs=pl.BlockSpec((_NG, _D), lambda i: (0, 0)),
        out_shape=jax.ShapeDtypeStruct((_NG, _D), jnp.float32),
    )(state, batch2, bih, bhh, emax, denom)


def _tc_final(rvec, ghead, J1, jb1, J2, jb2, S1, sb1, S2, sb2,
              bih, bhh, Wout, bout):
    def body(rv_ref, gh_ref, j1_ref, jb1_ref, j2_ref, jb2_ref,
             s1_ref, sb1_ref, s2_ref, sb2_ref, bi_ref, bh_ref,
             wo_ref, bo_ref, fin_ref, jb_ref, st_ref):
        hl = _hl_from_biases(bi_ref[...], bh_ref[...])
        q = jnp.broadcast_to(hl, (_NG, _D))
        qstar = jnp.concatenate([q, rv_ref[...]], axis=1)
        fin_ref[...] = jax.lax.dot_general(
            qstar, wo_ref[...], (((1,), (0,)), ((), ())),
            preferred_element_type=jnp.float32) + bo_ref[...]
        gj = gh_ref[:2 * _NJ]
        t = _lrelu(jax.lax.dot_general(
            gj, j1_ref[...], (((1,), (0,)), ((), ())),
            preferred_element_type=jnp.float32) + jb1_ref[...])
        t = jax.lax.dot_general(
            t, j2_ref[...], (((1,), (0,)), ((), ())),
            preferred_element_type=jnp.float32) + jb2_ref[...]
        jb_ref[...] = 0.5 * (t[:_NJ] + t[_NJ:])
        gs = gh_ref[2 * _NJ:]
        u = _lrelu(jax.lax.dot_general(
            gs, s1_ref[...], (((1,), (0,)), ((), ())),
            preferred_element_type=jnp.float32) + sb1_ref[...])
        st_ref[...] = jax.lax.dot_general(
            u, s2_ref[...], (((1,), (0,)), ((), ())),
            preferred_element_type=jnp.float32) + sb2_ref[...]

    return pl.pallas_call(
        body,
        out_shape=[
            jax.ShapeDtypeStruct((_NG, 2), jnp.float32),
            jax.ShapeDtypeStruct((_NJ, 1), jnp.float32),
            jax.ShapeDtypeStruct((_NS, _NOPS), jnp.float32),
        ],
    )(rvec, ghead, J1, jb1.reshape(1, _D), J2, jb2.reshape(1, 1),
      S1, sb1.reshape(1, _D), S2, sb2.reshape(1, _NOPS),
      bih, bhh, Wout, bout.reshape(1, 2))


def kernel(x, edge_index, edge_attr, jbond_atmidx, stem_atmidx, batch,
           W0, b0, A1, c1, A2, c2, root, cb,
           gru_Wih, gru_Whh, gru_bih, gru_bhh,
           S1, sb1, S2, sb2, J1, jb1, J2, jb2,
           lstm_Wih, lstm_Whh, lstm_bih, lstm_bhh, Wout, bout):
    src = edge_index[0]
    dst = edge_index[1]
    A2f = A2.reshape(_D * _D, _D)
    c2r = c2.reshape(_D, _D)
    batch2 = batch.reshape(_N, 1)
    lbih = lstm_bih.reshape(1, 4 * _D)
    lbhh = lstm_bhh.reshape(1, 4 * _D)
    zeros_n = jnp.zeros((_N, _D), jnp.float32)
    ones_e = jnp.ones((_E, _D), jnp.float32)
    eye = jnp.eye(_D, dtype=jnp.bfloat16)
    rep_he = jnp.repeat(eye, _D, axis=1)
    rep_s = jnp.tile(eye, (1, _D))
    A2f16 = A2f.astype(jnp.bfloat16)
    c2r16 = c2r.astype(jnp.bfloat16)

    cntp = _sc_scatter(ones_e, dst, zeros_n)
    state, cnt_inv = _tc_prologue(x, W0, b0, cntp)

    for _ in range(6):
        s = _sc_gather(state, src)
        msg = _tc_msg(edge_attr, s, A1, c1, A2f16, c2r16, rep_he, rep_s)
        aggp = _sc_scatter(msg, dst, zeros_n)
        state = _tc_update(aggp, cnt_inv, state, root, cb,
                           gru_Wih, gru_Whh, gru_bih, gru_bhh)

    hidx = jnp.concatenate(
        [jbond_atmidx[:, 0], jbond_atmidx[:, 1], stem_atmidx])
    ghead = _sc_gather(state, hidx)

    emax = _tc_emax(state, batch2, lbih, lbhh)
    denom = _tc_denom(state, batch2, lbih, lbhh, emax)
    rvec = _tc_rvec(state, batch2, lbih, lbhh, emax, denom)
    final, jb, stem = _tc_final(rvec, ghead, J1, jb1, J2, jb2,
                                S1, sb1, S2, sb2, lbih, lbhh, Wout, bout)
    return final, jb.reshape(_NJ), stem
